# baseline (speedup 1.0000x reference)
import jax
import jax.numpy as jnp
from jax.experimental import pallas as pl
from jax.experimental.pallas import tpu as pltpu

B = 16384
NUM_NODES = 8
FEAT = 16
HID = 64
OUT = 128
EPS = 1e-5
TILE = 2048
NT = B // TILE


def _stage1(x_ref, w1e_ref, b1_ref, t1_ref, st_ref, acc):
    i = pl.program_id(0)

    @pl.when(i == 0)
    def _():
        acc[...] = jnp.zeros_like(acc)

    t1 = jnp.dot(x_ref[...], w1e_ref[...],
                 preferred_element_type=jnp.float32) + b1_ref[...]
    t1_ref[...] = t1
    acc[0:1, :] += jnp.sum(t1, axis=0, keepdims=True)
    acc[1:2, :] += jnp.sum(t1 * t1, axis=0, keepdims=True)
    st_ref[...] = acc[...]


def _stage2(t1_ref, st_ref, g1_ref, be1_ref, w2_ref, b2_ref,
            t2_ref, st2_ref, acc):
    i = pl.program_id(0)

    @pl.when(i == 0)
    def _():
        acc[...] = jnp.zeros_like(acc)

    mu = st_ref[0:1, :] * (1.0 / B)
    var = st_ref[1:2, :] * (1.0 / B) - mu * mu
    scale = g1_ref[...] * jax.lax.rsqrt(var + EPS)
    shift = be1_ref[...] - mu * scale
    a1 = jax.nn.relu(t1_ref[...] * scale + shift)
    t2 = jnp.dot(a1, w2_ref[...],
                 preferred_element_type=jnp.float32) + b2_ref[...]
    t2_ref[...] = t2
    acc[0:1, :] += jnp.sum(t2, axis=0, keepdims=True)
    acc[1:2, :] += jnp.sum(t2 * t2, axis=0, keepdims=True)
    st2_ref[...] = acc[...]


def _stage3(t2_ref, st_ref, g2_ref, be2_ref, wfc_ref, bfc_ref, out_ref):
    mu = st_ref[0:1, :] * (1.0 / B)
    var = st_ref[1:2, :] * (1.0 / B) - mu * mu
    scale = g2_ref[...] * jax.lax.rsqrt(var + EPS)
    shift = be2_ref[...] - mu * scale
    a2 = jax.nn.relu(t2_ref[...] * scale + shift)
    out_ref[...] = jnp.dot(a2, wfc_ref[...],
                           preferred_element_type=jnp.float32) + bfc_ref[...]


def _tile_spec(cols):
    return pl.BlockSpec((TILE, cols), lambda i: (i, 0))


def _pinned(rows, cols):
    return pl.BlockSpec((rows, cols), lambda i: (0, 0))


@jax.jit
def kernel(x, W1, b1, g1, be1, W2, b2, g2, be2, Wfc, bfc):
    w1e = jnp.tile(W1 / NUM_NODES, (NUM_NODES, 1))
    wfc_eff = Wfc[:HID] + Wfc[HID:]
    row = lambda v: v.reshape(1, -1)

    t1, st1 = pl.pallas_call(
        _stage1,
        grid=(NT,),
        in_specs=[_tile_spec(NUM_NODES * FEAT), _pinned(NUM_NODES * FEAT, HID),
                  _pinned(1, HID)],
        out_specs=[_tile_spec(HID), _pinned(8, HID)],
        out_shape=[jax.ShapeDtypeStruct((B, HID), jnp.float32),
                   jax.ShapeDtypeStruct((8, HID), jnp.float32)],
        scratch_shapes=[pltpu.VMEM((8, HID), jnp.float32)],
        compiler_params=pltpu.CompilerParams(
            dimension_semantics=("arbitrary",)),
    )(x, w1e, row(b1))

    t2, st2 = pl.pallas_call(
        _stage2,
        grid=(NT,),
        in_specs=[_tile_spec(HID), _pinned(8, HID), _pinned(1, HID),
                  _pinned(1, HID), _pinned(HID, HID), _pinned(1, HID)],
        out_specs=[_tile_spec(HID), _pinned(8, HID)],
        out_shape=[jax.ShapeDtypeStruct((B, HID), jnp.float32),
                   jax.ShapeDtypeStruct((8, HID), jnp.float32)],
        scratch_shapes=[pltpu.VMEM((8, HID), jnp.float32)],
        compiler_params=pltpu.CompilerParams(
            dimension_semantics=("arbitrary",)),
    )(t1, st1, row(g1), row(be1), W2, row(b2))

    return pl.pallas_call(
        _stage3,
        grid=(NT,),
        in_specs=[_tile_spec(HID), _pinned(8, HID), _pinned(1, HID),
                  _pinned(1, HID), _pinned(HID, OUT), _pinned(1, OUT)],
        out_specs=_tile_spec(OUT),
        out_shape=jax.ShapeDtypeStruct((B, OUT), jnp.float32),
        compiler_params=pltpu.CompilerParams(
            dimension_semantics=("arbitrary",)),
    )(t2, st2, row(g2), row(be2), wfc_eff, row(bfc))


# speedup vs baseline: 1.9315x; 1.9315x over previous
import jax
import jax.numpy as jnp
from jax.experimental import pallas as pl
from jax.experimental.pallas import tpu as pltpu

B = 16384
NUM_NODES = 8
FEAT = 16
HID = 64
OUT = 128
EPS = 1e-5
TILE = 4096
NT = B // TILE
PREC = jax.lax.Precision.DEFAULT


def _fused_kernel(x_ref, w1e_ref, b1_ref, g1_ref, be1_ref,
                  w2_ref, b2_ref, g2_ref, be2_ref,
                  wfc_ref, bfc_ref, out_ref,
                  t1_s, t2_s, s1_s, s2_s):
    p = pl.program_id(0)
    i = pl.program_id(1)
    rows = pl.ds(i * TILE, TILE)

    @pl.when(p == 0)
    def _phase0():
        @pl.when(i == 0)
        def _():
            s1_s[...] = jnp.zeros_like(s1_s)

        t1 = jnp.dot(x_ref[...], w1e_ref[...],
                     preferred_element_type=jnp.float32,
                     precision=PREC) + b1_ref[...]
        t1_s[rows, :] = t1
        s1_s[0:1, :] += jnp.sum(t1, axis=0, keepdims=True)
        s1_s[1:2, :] += jnp.sum(t1 * t1, axis=0, keepdims=True)

    @pl.when(p == 1)
    def _phase1():
        @pl.when(i == 0)
        def _():
            s2_s[...] = jnp.zeros_like(s2_s)

        mu = s1_s[0:1, :] * (1.0 / B)
        var = s1_s[1:2, :] * (1.0 / B) - mu * mu
        scale = g1_ref[...] * jax.lax.rsqrt(var + EPS)
        t1 = t1_s[rows, :]
        a1 = jax.nn.relu((t1 - mu) * scale + be1_ref[...])
        t2 = jnp.dot(a1, w2_ref[...],
                     preferred_element_type=jnp.float32,
                     precision=PREC) + b2_ref[...]
        t2_s[rows, :] = t2
        s2_s[0:1, :] += jnp.sum(t2, axis=0, keepdims=True)
        s2_s[1:2, :] += jnp.sum(t2 * t2, axis=0, keepdims=True)

    @pl.when(p == 2)
    def _phase2():
        mu = s2_s[0:1, :] * (1.0 / B)
        var = s2_s[1:2, :] * (1.0 / B) - mu * mu
        scale = g2_ref[...] * jax.lax.rsqrt(var + EPS)
        t2 = t2_s[rows, :]
        a2 = jax.nn.relu((t2 - mu) * scale + be2_ref[...])
        out_ref[...] = jnp.dot(a2, wfc_ref[...],
                               preferred_element_type=jnp.float32,
                               precision=PREC) + bfc_ref[...]


@jax.jit
def kernel(x, W1, b1, g1, be1, W2, b2, g2, be2, Wfc, bfc):
    w1e = jnp.tile(W1 / NUM_NODES, (NUM_NODES, 1))
    wfc_eff = Wfc[:HID] + Wfc[HID:]
    row = lambda v: v.reshape(1, -1)

    pinned0 = lambda p, i: (0, 0)
    grid_spec = pltpu.PrefetchScalarGridSpec(
        num_scalar_prefetch=0,
        grid=(3, NT),
        in_specs=[
            pl.BlockSpec((TILE, NUM_NODES * FEAT),
                         lambda p, i: (jnp.where(p == 0, i, 0), 0)),
            pl.BlockSpec((NUM_NODES * FEAT, HID), pinned0),
            pl.BlockSpec((1, HID), pinned0),
            pl.BlockSpec((1, HID), pinned0),
            pl.BlockSpec((1, HID), pinned0),
            pl.BlockSpec((HID, HID), pinned0),
            pl.BlockSpec((1, HID), pinned0),
            pl.BlockSpec((1, HID), pinned0),
            pl.BlockSpec((1, HID), pinned0),
            pl.BlockSpec((HID, OUT), pinned0),
            pl.BlockSpec((1, OUT), pinned0),
        ],
        out_specs=pl.BlockSpec((TILE, OUT),
                               lambda p, i: (jnp.where(p == 2, i, 0), 0)),
        scratch_shapes=[
            pltpu.VMEM((B, HID), jnp.float32),
            pltpu.VMEM((B, HID), jnp.float32),
            pltpu.VMEM((8, HID), jnp.float32),
            pltpu.VMEM((8, HID), jnp.float32),
        ],
    )
    return pl.pallas_call(
        _fused_kernel,
        grid_spec=grid_spec,
        out_shape=jax.ShapeDtypeStruct((B, OUT), jnp.float32),
        compiler_params=pltpu.CompilerParams(
            dimension_semantics=("arbitrary", "arbitrary"),
        ),
    )(x, w1e, row(b1), row(g1), row(be1),
      W2, row(b2), row(g2), row(be2), wfc_eff, row(bfc))


# speedup vs baseline: 2.0597x; 1.0664x over previous
import jax
import jax.numpy as jnp
from jax.experimental import pallas as pl
from jax.experimental.pallas import tpu as pltpu

B = 16384
NUM_NODES = 8
FEAT = 16
HID = 64
OUT = 128
EPS = 1e-5
TILE = 8192
NT = B // TILE
PREC = jax.lax.Precision.DEFAULT


def _fused_kernel(x_ref, w1e_ref, b1_ref, g1_ref, be1_ref,
                  w2_ref, b2_ref, g2_ref, be2_ref,
                  wfc_ref, bfc_ref, out_ref,
                  t1_s, t2_s, s1_s, s2_s):
    p = pl.program_id(0)
    i = pl.program_id(1)
    rows = pl.ds(i * TILE, TILE)

    @pl.when(p == 0)
    def _phase0():
        @pl.when(i == 0)
        def _():
            s1_s[...] = jnp.zeros_like(s1_s)

        t1 = jnp.dot(x_ref[...], w1e_ref[...],
                     preferred_element_type=jnp.float32,
                     precision=PREC) + b1_ref[...]
        t1_s[rows, :] = t1
        s1_s[0:1, :] += jnp.sum(t1, axis=0, keepdims=True)
        s1_s[1:2, :] += jnp.sum(t1 * t1, axis=0, keepdims=True)

    @pl.when(p == 1)
    def _phase1():
        @pl.when(i == 0)
        def _():
            s2_s[...] = jnp.zeros_like(s2_s)

        mu = s1_s[0:1, :] * (1.0 / B)
        var = s1_s[1:2, :] * (1.0 / B) - mu * mu
        scale = g1_ref[...] * jax.lax.rsqrt(var + EPS)
        t1 = t1_s[rows, :]
        a1 = jax.nn.relu((t1 - mu) * scale + be1_ref[...])
        t2 = jnp.dot(a1, w2_ref[...],
                     preferred_element_type=jnp.float32,
                     precision=PREC) + b2_ref[...]
        t2_s[rows, :] = t2
        s2_s[0:1, :] += jnp.sum(t2, axis=0, keepdims=True)
        s2_s[1:2, :] += jnp.sum(t2 * t2, axis=0, keepdims=True)

    @pl.when(p == 2)
    def _phase2():
        mu = s2_s[0:1, :] * (1.0 / B)
        var = s2_s[1:2, :] * (1.0 / B) - mu * mu
        scale = g2_ref[...] * jax.lax.rsqrt(var + EPS)
        t2 = t2_s[rows, :]
        a2 = jax.nn.relu((t2 - mu) * scale + be2_ref[...])
        out_ref[...] = jnp.dot(a2, wfc_ref[...],
                               preferred_element_type=jnp.float32,
                               precision=PREC) + bfc_ref[...]


@jax.jit
def kernel(x, W1, b1, g1, be1, W2, b2, g2, be2, Wfc, bfc):
    w1e = jnp.tile(W1 / NUM_NODES, (NUM_NODES, 1))
    wfc_eff = Wfc[:HID] + Wfc[HID:]
    row = lambda v: v.reshape(1, -1)

    pinned0 = lambda p, i: (0, 0)
    grid_spec = pltpu.PrefetchScalarGridSpec(
        num_scalar_prefetch=0,
        grid=(3, NT),
        in_specs=[
            pl.BlockSpec((TILE, NUM_NODES * FEAT),
                         lambda p, i: (jnp.where(p == 0, i, 0), 0)),
            pl.BlockSpec((NUM_NODES * FEAT, HID), pinned0),
            pl.BlockSpec((1, HID), pinned0),
            pl.BlockSpec((1, HID), pinned0),
            pl.BlockSpec((1, HID), pinned0),
            pl.BlockSpec((HID, HID), pinned0),
            pl.BlockSpec((1, HID), pinned0),
            pl.BlockSpec((1, HID), pinned0),
            pl.BlockSpec((1, HID), pinned0),
            pl.BlockSpec((HID, OUT), pinned0),
            pl.BlockSpec((1, OUT), pinned0),
        ],
        out_specs=pl.BlockSpec((TILE, OUT),
                               lambda p, i: (jnp.where(p == 2, i, 0), 0)),
        scratch_shapes=[
            pltpu.VMEM((B, HID), jnp.float32),
            pltpu.VMEM((B, HID), jnp.float32),
            pltpu.VMEM((8, HID), jnp.float32),
            pltpu.VMEM((8, HID), jnp.float32),
        ],
    )
    return pl.pallas_call(
        _fused_kernel,
        grid_spec=grid_spec,
        out_shape=jax.ShapeDtypeStruct((B, OUT), jnp.float32),
        compiler_params=pltpu.CompilerParams(
            dimension_semantics=("arbitrary", "arbitrary"),
        ),
    )(x, w1e, row(b1), row(g1), row(be1),
      W2, row(b2), row(g2), row(be2), wfc_eff, row(bfc))
